# Initial kernel scaffold; baseline (speedup 1.0000x reference)
import jax
import jax.numpy as jnp
from jax.experimental import pallas as pl


def kernel(xyz_low, xyz_high, feat_low, feat_high_skip, W, b):
    raise NotImplementedError("write your pallas kernel here")


# speedup vs baseline: 18.8719x; 18.8719x over previous
import jax
import jax.numpy as jnp
from jax.experimental import pallas as pl
from jax.experimental.pallas import tpu as pltpu

_K = 3
_TQ = 256


def _fused_body(xh_ref, xl_ref, f_ref, sk_ref, w_ref, b_ref, o_ref, g_ref):
    j = pl.program_id(1)
    c_low = f_ref.shape[1]

    @pl.when(j == 0)
    def _build_g():
        w1 = w_ref[:, :c_low]
        g_ref[:, :] = jax.lax.dot_general(
            w1, f_ref[0], (((1,), (0,)), ((), ())),
            preferred_element_type=jnp.float32,
            precision=jax.lax.Precision.HIGHEST)

    q = xh_ref[0]
    r = xl_ref[0]
    q2 = jnp.sum(q * q, axis=1, keepdims=True)
    r2 = jnp.sum(r * r, axis=0, keepdims=True)
    qr = jax.lax.dot_general(
        q.astype(jnp.bfloat16), r.astype(jnp.bfloat16),
        (((1,), (0,)), ((), ())),
        preferred_element_type=jnp.float32)
    d2 = q2 + r2 - 2.0 * qr
    dist = jnp.sqrt(jnp.maximum(d2, 0.0))

    n2 = dist.shape[1]
    iota = jax.lax.broadcasted_iota(jnp.int32, dist.shape, 1)
    s = jnp.zeros(dist.shape, jnp.float32)
    total = jnp.zeros((dist.shape[0], 1), jnp.float32)
    for _ in range(_K):
        m = jnp.min(dist, axis=1, keepdims=True)
        col = jnp.min(jnp.where(dist == m, iota, n2), axis=1, keepdims=True)
        sel = iota == col
        wk = 1.0 / jnp.maximum(m, 1e-8)
        s = s + jnp.where(sel, wk, 0.0)
        total = total + wk
        dist = jnp.where(sel, jnp.inf, dist)
    s = s / total

    interp = jax.lax.dot_general(s, g_ref[:, :], (((1,), (1,)), ((), ())),
                                 preferred_element_type=jnp.float32,
                                 precision=jax.lax.Precision.HIGHEST)
    w2 = w_ref[:, c_low:]
    skp = jax.lax.dot_general(sk_ref[0], w2, (((1,), (1,)), ((), ())),
                              preferred_element_type=jnp.float32,
                              precision=jax.lax.Precision.HIGHEST)
    o_ref[0] = interp + skp + b_ref[:, :]


def kernel(xyz_low, xyz_high, feat_low, feat_high_skip, W, b):
    B, N1, _ = xyz_high.shape
    N2 = xyz_low.shape[1]
    c_low = feat_low.shape[1]
    c_skip = feat_high_skip.shape[2]
    out_dim = W.shape[0]

    xh = jnp.concatenate(
        [xyz_high, jnp.zeros((B, N1, 5), xyz_high.dtype)], axis=-1)
    xl = jnp.concatenate(
        [xyz_low, jnp.zeros((B, N2, 5), xyz_low.dtype)], axis=-1)
    xl = jnp.swapaxes(xl, 1, 2)
    b2 = b.reshape(1, out_dim)

    return pl.pallas_call(
        _fused_body,
        grid=(B, N1 // _TQ),
        in_specs=[
            pl.BlockSpec((1, _TQ, 8), lambda bi, j: (bi, j, 0)),
            pl.BlockSpec((1, 8, N2), lambda bi, j: (bi, 0, 0)),
            pl.BlockSpec((1, c_low, N2), lambda bi, j: (bi, 0, 0)),
            pl.BlockSpec((1, _TQ, c_skip), lambda bi, j: (bi, j, 0)),
            pl.BlockSpec((out_dim, c_low + c_skip), lambda bi, j: (0, 0)),
            pl.BlockSpec((1, out_dim), lambda bi, j: (0, 0)),
        ],
        out_specs=pl.BlockSpec((1, _TQ, out_dim), lambda bi, j: (bi, j, 0)),
        out_shape=jax.ShapeDtypeStruct((B, N1, out_dim), jnp.float32),
        scratch_shapes=[pltpu.VMEM((out_dim, N2), jnp.float32)],
    )(xh, xl, feat_low, feat_high_skip, W, b2)


# speedup vs baseline: 29.4915x; 1.5627x over previous
import jax
import jax.numpy as jnp
from jax.experimental import pallas as pl
from jax.experimental.pallas import tpu as pltpu

_K = 3
_TQ = 256


def _fused_body(xh_ref, xl_ref, f_ref, sk_ref, w_ref, b_ref, o_ref, g_ref):
    j = pl.program_id(1)
    c_low = f_ref.shape[1]

    @pl.when(j == 0)
    def _build_g():
        w1 = w_ref[:, :c_low]
        g_ref[:, :] = jax.lax.dot_general(
            w1, f_ref[0], (((1,), (0,)), ((), ())),
            preferred_element_type=jnp.float32,
            precision=jax.lax.Precision.HIGHEST)

    q = xh_ref[0]
    r = xl_ref[0]
    q2 = jnp.sum(q * q, axis=1, keepdims=True)
    r2 = jnp.sum(r * r, axis=0, keepdims=True)
    qr = jax.lax.dot_general(
        q.astype(jnp.bfloat16), r.astype(jnp.bfloat16),
        (((1,), (0,)), ((), ())),
        preferred_element_type=jnp.float32)
    d2 = q2 + r2 - 2.0 * qr
    dist = jnp.sqrt(jnp.maximum(d2, 0.0))

    iota = jax.lax.broadcasted_iota(
        jnp.int32, dist.shape, 1).astype(jnp.float32)
    big = jnp.float32(1e9)
    s = jnp.zeros(dist.shape, jnp.float32)
    total = jnp.zeros((dist.shape[0], 1), jnp.float32)
    for k in range(_K):
        m = jnp.min(dist, axis=1, keepdims=True)
        col = jnp.min(jnp.where(dist == m, iota, big), axis=1, keepdims=True)
        sel = iota == col
        wk = 1.0 / jnp.maximum(m, 1e-8)
        s = jnp.where(sel, jnp.broadcast_to(wk, s.shape), s)
        total = total + wk
        if k + 1 < _K:
            dist = jnp.where(sel, jnp.inf, dist)
    s = s * (1.0 / total)

    interp = jax.lax.dot_general(s.astype(jnp.bfloat16),
                                 g_ref[:, :].astype(jnp.bfloat16),
                                 (((1,), (1,)), ((), ())),
                                 preferred_element_type=jnp.float32)
    w2 = w_ref[:, c_low:]
    skp = jax.lax.dot_general(sk_ref[0], w2, (((1,), (1,)), ((), ())),
                              preferred_element_type=jnp.float32,
                              precision=jax.lax.Precision.HIGHEST)
    o_ref[0] = interp + skp + b_ref[:, :]


def kernel(xyz_low, xyz_high, feat_low, feat_high_skip, W, b):
    B, N1, _ = xyz_high.shape
    N2 = xyz_low.shape[1]
    c_low = feat_low.shape[1]
    c_skip = feat_high_skip.shape[2]
    out_dim = W.shape[0]

    xh = jnp.concatenate(
        [xyz_high, jnp.zeros((B, N1, 5), xyz_high.dtype)], axis=-1)
    xl = jnp.concatenate(
        [xyz_low, jnp.zeros((B, N2, 5), xyz_low.dtype)], axis=-1)
    xl = jnp.swapaxes(xl, 1, 2)
    b2 = b.reshape(1, out_dim)

    return pl.pallas_call(
        _fused_body,
        grid=(B, N1 // _TQ),
        in_specs=[
            pl.BlockSpec((1, _TQ, 8), lambda bi, j: (bi, j, 0)),
            pl.BlockSpec((1, 8, N2), lambda bi, j: (bi, 0, 0)),
            pl.BlockSpec((1, c_low, N2), lambda bi, j: (bi, 0, 0)),
            pl.BlockSpec((1, _TQ, c_skip), lambda bi, j: (bi, j, 0)),
            pl.BlockSpec((out_dim, c_low + c_skip), lambda bi, j: (0, 0)),
            pl.BlockSpec((1, out_dim), lambda bi, j: (0, 0)),
        ],
        out_specs=pl.BlockSpec((1, _TQ, out_dim), lambda bi, j: (bi, j, 0)),
        out_shape=jax.ShapeDtypeStruct((B, N1, out_dim), jnp.float32),
        scratch_shapes=[pltpu.VMEM((out_dim, N2), jnp.float32)],
    )(xh, xl, feat_low, feat_high_skip, W, b2)
